# baseline (device time: 43215 ns/iter reference)
import jax
import jax.numpy as jnp
from jax import lax
from jax.experimental import pallas as pl
from jax.experimental.pallas import tpu as pltpu

N_DEV = 8
N_EXP = 32


def kernel(x, router_W, route_idx, expert_W):
    n_tok, d = x.shape
    n_loc, _, h = expert_W.shape

    def body(x_ref, rw_ref, idx_ref, ew_ref, out_ref, comm_ref,
             cw_send, cw_recv, ccw_send, ccw_recv, f_send, f_recv):
        my = lax.axis_index("i")

        def ring2dev(i):
            return i ^ jnp.where(i >= 4, 3, 0)

        p = ring2dev(my)
        right = ring2dev((p + 1) % N_DEV)
        left = ring2dev((p - 1) % N_DEV)
        is_even = (p % 2) == 0
        q = (p + jnp.where(is_even, 3, 5)) % N_DEV
        free = ring2dev(q)

        barrier_sem = pltpu.get_barrier_semaphore()
        for nbr in (left, right, free):
            pl.semaphore_signal(
                barrier_sem, inc=1,
                device_id=(nbr,), device_id_type=pl.DeviceIdType.MESH,
            )
        pl.semaphore_wait(barrier_sem, 3)

        comm_ref[0] = ew_ref[...].astype(jnp.bfloat16)

        def rc(src_slot, dst_slot, sends, recvs, s, dev):
            return pltpu.make_async_remote_copy(
                src_ref=comm_ref.at[src_slot], dst_ref=comm_ref.at[dst_slot],
                send_sem=sends.at[s], recv_sem=recvs.at[s],
                device_id=(dev,), device_id_type=pl.DeviceIdType.MESH,
            )

        def rc_half(src_slot, lo, dst_slot, sends, recvs, s, dev):
            return pltpu.make_async_remote_copy(
                src_ref=comm_ref.at[src_slot, pl.ds(lo, 2)],
                dst_ref=comm_ref.at[dst_slot, pl.ds(lo, 2)],
                send_sem=sends.at[s], recv_sem=recvs.at[s],
                device_id=(dev,), device_id_type=pl.DeviceIdType.MESH,
            )


        cw1a = rc_half(0, 0, 1, cw_send, cw_recv, 0, right)
        cw1a.start()
        cw1b = rc_half(0, 2, 1, cw_send, cw_recv, 1, right)
        cw1b.start()
        ccw1a = rc_half(0, 0, 7, ccw_send, ccw_recv, 0, left)
        ccw1a.start()
        ccw1b = rc_half(0, 2, 7, ccw_send, ccw_recv, 1, left)
        ccw1b.start()

        @pl.when(is_even)
        def _():
            rc(0, 3, f_send, f_recv, 0, free).start()

        @pl.when(~is_even)
        def _():
            rc(0, 5, f_send, f_recv, 0, free).start()

        xf = x_ref[...]
        scores = jnp.dot(xf, rw_ref[...], preferred_element_type=jnp.float32)
        mx = jnp.max(scores, axis=-1, keepdims=True)
        pe = jnp.exp(scores - mx)
        pe = pe / jnp.sum(pe, axis=-1, keepdims=True)
        eids = lax.broadcasted_iota(jnp.int32, (n_tok, N_EXP), 1)
        routed = (eids == idx_ref[:, 0:1]) | (eids == idx_ref[:, 1:2])
        pr = jnp.where(routed, pe, 0.0)
        gates = pr / jnp.sum(pr, axis=-1, keepdims=True)

        def gcol(eid):
            return jnp.sum(jnp.where(eids == eid, gates, 0.0),
                           axis=-1, keepdims=True)

        def chalf(slot, lo):
            origin = ring2dev((p - slot) % N_DEV)
            xg = jnp.concatenate(
                [(xf * gcol(origin * n_loc + lo + j)).astype(jnp.bfloat16)
                 for j in range(2)], axis=1)
            w = jnp.concatenate([comm_ref[slot, lo], comm_ref[slot, lo + 1]],
                                axis=0)
            return jnp.dot(xg, w, preferred_element_type=jnp.float32)

        def cfull_dyn(slot):
            origin = ring2dev((p - slot) % N_DEV)
            xg = jnp.concatenate(
                [(xf * gcol(origin * n_loc + j)).astype(jnp.bfloat16)
                 for j in range(n_loc)], axis=1)
            w = comm_ref[pl.ds(slot, 1)].reshape(n_loc * d, h)
            return jnp.dot(xg, w, preferred_element_type=jnp.float32)

        acc = chalf(0, 0) + chalf(0, 2)

        cw1a.wait()
        rc_half(1, 0, 2, cw_send, cw_recv, 2, right).start()

        @pl.when(is_even)
        def _():
            rc_half(1, 0, 4, f_send, f_recv, 1, free).start()

        acc += chalf(1, 0)
        ccw1a.wait()
        rc_half(7, 0, 6, ccw_send, ccw_recv, 2, left).start()

        @pl.when(~is_even)
        def _():
            rc_half(7, 0, 4, f_send, f_recv, 1, free).start()

        acc += chalf(7, 0)
        cw1b.wait()
        rc_half(1, 2, 2, cw_send, cw_recv, 3, right).start()

        @pl.when(is_even)
        def _():
            rc_half(1, 2, 4, f_send, f_recv, 2, free).start()

        acc += chalf(1, 2)
        ccw1b.wait()
        rc_half(7, 2, 6, ccw_send, ccw_recv, 3, left).start()

        @pl.when(~is_even)
        def _():
            rc_half(7, 2, 4, f_send, f_recv, 2, free).start()

        acc += chalf(7, 2)

        rc(0, 4, f_send, f_recv, 0, free).wait()
        acc += cfull_dyn(jnp.where(is_even, 5, 3))

        rc_half(1, 0, 2, cw_send, cw_recv, 2, left).wait()

        @pl.when(is_even)
        def _():
            rc_half(2, 0, 5, f_send, f_recv, 3, free).start()

        acc += chalf(2, 0)
        rc_half(7, 0, 6, ccw_send, ccw_recv, 2, right).wait()

        @pl.when(~is_even)
        def _():
            rc_half(6, 0, 3, f_send, f_recv, 3, free).start()

        acc += chalf(6, 0)
        rc_half(1, 2, 2, cw_send, cw_recv, 3, left).wait()

        @pl.when(~is_even)
        def _():
            rc_half(2, 2, 3, cw_send, cw_recv, 4, right).start()

        acc += chalf(2, 2)
        rc_half(7, 2, 6, ccw_send, ccw_recv, 3, right).wait()

        @pl.when(is_even)
        def _():
            rc_half(6, 2, 5, ccw_send, ccw_recv, 4, left).start()

        acc += chalf(6, 2)

        rc_half(1, 0, 4, f_send, f_recv, 1, free).wait()
        acc += chalf(4, 0)
        rc_half(1, 2, 4, f_send, f_recv, 2, free).wait()
        acc += chalf(4, 2)

        rc_half(2, 0, 3, f_send, f_recv, 3, free).wait()

        @pl.when(is_even)
        def _():
            rc_half(2, 2, 3, cw_send, cw_recv, 4, left).wait_recv()
            rc_half(6, 2, 5, ccw_send, ccw_recv, 4, left).wait_send()

        @pl.when(~is_even)
        def _():
            rc_half(6, 2, 5, ccw_send, ccw_recv, 4, right).wait_recv()
            rc_half(2, 2, 3, cw_send, cw_recv, 4, right).wait_send()

        acc += cfull_dyn(jnp.where(is_even, 3, 5))
        out_ref[...] = acc

    return pl.pallas_call(
        body,
        out_shape=jax.ShapeDtypeStruct((n_tok, h), jnp.float32),
        in_specs=[pl.BlockSpec(memory_space=pltpu.VMEM)] * 4,
        out_specs=pl.BlockSpec(memory_space=pltpu.VMEM),
        scratch_shapes=[
            pltpu.VMEM((N_DEV, n_loc, d, h), jnp.bfloat16),
            pltpu.SemaphoreType.DMA((5,)),
            pltpu.SemaphoreType.DMA((5,)),
            pltpu.SemaphoreType.DMA((5,)),
            pltpu.SemaphoreType.DMA((5,)),
            pltpu.SemaphoreType.DMA((4,)),
            pltpu.SemaphoreType.DMA((4,)),
        ],
        compiler_params=pltpu.CompilerParams(collective_id=0),
    )(x, router_W, route_idx, expert_W)


# device time: 42338 ns/iter; 1.0207x vs baseline; 1.0207x over previous
import jax
import jax.numpy as jnp
from jax import lax
from jax.experimental import pallas as pl
from jax.experimental.pallas import tpu as pltpu

N_DEV = 8
N_EXP = 32


def kernel(x, router_W, route_idx, expert_W):
    n_tok, d = x.shape
    n_loc, _, h = expert_W.shape

    def body(x_ref, rw_ref, idx_ref, ew_ref, out_ref, comm_ref,
             cw_send, cw_recv, ccw_send, ccw_recv, f_send, f_recv):
        my = lax.axis_index("i")

        def ring2dev(i):
            return i ^ jnp.where(i >= 4, 3, 0)

        p = ring2dev(my)
        right = ring2dev((p + 1) % N_DEV)
        left = ring2dev((p - 1) % N_DEV)
        is_even = (p % 2) == 0
        q = (p + jnp.where(is_even, 3, 5)) % N_DEV
        free = ring2dev(q)

        barrier_sem = pltpu.get_barrier_semaphore()
        for nbr in (left, right, free):
            pl.semaphore_signal(
                barrier_sem, inc=1,
                device_id=(nbr,), device_id_type=pl.DeviceIdType.MESH,
            )
        pl.semaphore_wait(barrier_sem, 3)

        comm_ref[0] = ew_ref[...].astype(jnp.bfloat16)

        def rc_half(src_slot, lo, dst_slot, sends, recvs, s, dev):
            return pltpu.make_async_remote_copy(
                src_ref=comm_ref.at[src_slot, pl.ds(lo, 2)],
                dst_ref=comm_ref.at[dst_slot, pl.ds(lo, 2)],
                send_sem=sends.at[s], recv_sem=recvs.at[s],
                device_id=(dev,), device_id_type=pl.DeviceIdType.MESH,
            )


        cw1a = rc_half(0, 0, 1, cw_send, cw_recv, 0, right)
        cw1a.start()
        cw1b = rc_half(0, 2, 1, cw_send, cw_recv, 1, right)
        cw1b.start()
        ccw1a = rc_half(0, 0, 7, ccw_send, ccw_recv, 0, left)
        ccw1a.start()
        ccw1b = rc_half(0, 2, 7, ccw_send, ccw_recv, 1, left)
        ccw1b.start()

        @pl.when(is_even)
        def _():
            rc_half(0, 0, 3, f_send, f_recv, 0, free).start()

        @pl.when(~is_even)
        def _():
            rc_half(0, 0, 5, f_send, f_recv, 0, free).start()

        xf = x_ref[...]
        scores = jnp.dot(xf, rw_ref[...], preferred_element_type=jnp.float32)
        mx = jnp.max(scores, axis=-1, keepdims=True)
        pe = jnp.exp(scores - mx)
        pe = pe / jnp.sum(pe, axis=-1, keepdims=True)
        eids = lax.broadcasted_iota(jnp.int32, (n_tok, N_EXP), 1)
        routed = (eids == idx_ref[:, 0:1]) | (eids == idx_ref[:, 1:2])
        pr = jnp.where(routed, pe, 0.0)
        gates = pr / jnp.sum(pr, axis=-1, keepdims=True)

        def gcol(eid):
            return jnp.sum(jnp.where(eids == eid, gates, 0.0),
                           axis=-1, keepdims=True)

        def xg2(origin, lo):
            return jnp.concatenate(
                [(xf * gcol(origin * n_loc + lo + j)).astype(jnp.bfloat16)
                 for j in range(2)], axis=1)

        def chalf(slot, lo):
            origin = ring2dev((p - slot) % N_DEV)
            w = jnp.concatenate([comm_ref[slot, lo], comm_ref[slot, lo + 1]],
                                axis=0)
            return jnp.dot(xg2(origin, lo), w,
                           preferred_element_type=jnp.float32)

        def chalf_dyn(slot, lo):
            origin = ring2dev((p - slot) % N_DEV)
            w = comm_ref[pl.ds(slot, 1), pl.ds(lo, 2)].reshape(2 * d, h)
            return jnp.dot(xg2(origin, lo), w,
                           preferred_element_type=jnp.float32)

        acc = chalf(0, 0) + chalf(0, 2)

        cw1a.wait()
        rc_half(1, 0, 2, cw_send, cw_recv, 2, right).start()

        @pl.when(is_even)
        def _():
            rc_half(1, 0, 4, f_send, f_recv, 1, free).start()

        acc += chalf(1, 0)
        ccw1a.wait()
        rc_half(7, 0, 6, ccw_send, ccw_recv, 2, left).start()

        @pl.when(~is_even)
        def _():
            rc_half(7, 0, 4, f_send, f_recv, 1, free).start()

        acc += chalf(7, 0)
        cw1b.wait()
        rc_half(1, 2, 2, cw_send, cw_recv, 3, right).start()

        @pl.when(is_even)
        def _():
            rc_half(1, 2, 4, f_send, f_recv, 2, free).start()

        acc += chalf(1, 2)
        ccw1b.wait()
        rc_half(7, 2, 6, ccw_send, ccw_recv, 3, left).start()

        @pl.when(~is_even)
        def _():
            rc_half(7, 2, 4, f_send, f_recv, 2, free).start()

        acc += chalf(7, 2)

        rc_half(0, 0, 4, f_send, f_recv, 0, free).wait()
        acc += chalf_dyn(jnp.where(is_even, 5, 3), 0)

        rc_half(1, 0, 2, cw_send, cw_recv, 2, left).wait()

        @pl.when(is_even)
        def _():
            rc_half(2, 0, 5, f_send, f_recv, 3, free).start()

        acc += chalf(2, 0)
        rc_half(7, 0, 6, ccw_send, ccw_recv, 2, right).wait()

        @pl.when(~is_even)
        def _():
            rc_half(6, 0, 3, f_send, f_recv, 3, free).start()

        acc += chalf(6, 0)
        rc_half(1, 2, 2, cw_send, cw_recv, 3, left).wait()
        cw3 = rc_half(2, 2, 3, cw_send, cw_recv, 4, right)
        cw3.start()
        acc += chalf(2, 2)
        rc_half(7, 2, 6, ccw_send, ccw_recv, 3, right).wait()
        ccw3 = rc_half(6, 2, 5, ccw_send, ccw_recv, 4, left)
        ccw3.start()
        acc += chalf(6, 2)

        rc_half(1, 0, 4, f_send, f_recv, 1, free).wait()
        acc += chalf(4, 0)
        rc_half(1, 2, 4, f_send, f_recv, 2, free).wait()
        acc += chalf(4, 2)

        cw3.wait()
        acc += chalf(3, 2)
        ccw3.wait()
        acc += chalf(5, 2)

        rc_half(2, 0, 3, f_send, f_recv, 3, free).wait()
        acc += chalf_dyn(jnp.where(is_even, 3, 5), 0)

        out_ref[...] = acc

    return pl.pallas_call(
        body,
        out_shape=jax.ShapeDtypeStruct((n_tok, h), jnp.float32),
        in_specs=[pl.BlockSpec(memory_space=pltpu.VMEM)] * 4,
        out_specs=pl.BlockSpec(memory_space=pltpu.VMEM),
        scratch_shapes=[
            pltpu.VMEM((N_DEV, n_loc, d, h), jnp.bfloat16),
            pltpu.SemaphoreType.DMA((5,)),
            pltpu.SemaphoreType.DMA((5,)),
            pltpu.SemaphoreType.DMA((5,)),
            pltpu.SemaphoreType.DMA((5,)),
            pltpu.SemaphoreType.DMA((4,)),
            pltpu.SemaphoreType.DMA((4,)),
        ],
        compiler_params=pltpu.CompilerParams(collective_id=0),
    )(x, router_W, route_idx, expert_W)
